# baseline (speedup 1.0000x reference)
import functools

import jax
import jax.numpy as jnp
from jax import lax
from jax.experimental import pallas as pl
from jax.experimental.pallas import tpu as pltpu
from jax.experimental.pallas import tpu_sc as plsc

N = 10000
E = 320000
IN_C = 128
HID = 16
OUT_C = 64
K = 10
ALPHA = 0.1

NC = 2
NS = 16
W = NC * NS
C = 128
CH = 80
EPW = C * CH
E_PAD = W * EPW
NP = 10112
RPT = NP // NS

_mesh = plsc.VectorSubcoreMesh(
    core_axis_name="c", subcore_axis_name="s", num_cores=NC, num_subcores=NS
)


def _mlp_body(x_ref, w1_ref, b1_ref, w2_ref, b2_ref, h_ref):
    h1 = jnp.dot(x_ref[...], w1_ref[...], preferred_element_type=jnp.float32,
                 precision=lax.Precision.HIGHEST)
    h1 = jnp.maximum(h1 + b1_ref[...], 0.0)
    h2 = jnp.dot(h1, w2_ref[...], preferred_element_type=jnp.float32,
                 precision=lax.Precision.HIGHEST)
    h_ref[...] = h2 + b2_ref[...]


def _prep_body(deg16_ref, h_ref, dinv_ref, dinv2_ref, p0_ref):
    degs = deg16_ref[0] + deg16_ref[1]
    deg = jnp.sum(degs, axis=1, keepdims=True) * (1.0 / 16.0) + 1.0
    rows = lax.broadcasted_iota(jnp.int32, (NP, 1), 0)
    dinv = jnp.where(rows < N, lax.rsqrt(deg), 0.0)
    dinv_ref[...] = dinv
    dinv2_ref[...] = dinv * dinv
    p0_ref[...] = dinv * h_ref[...]


def _blend_body(agg_ref, out_ref, h_ref, dinv_ref, dinv2_ref,
                out_next_ref, p_next_ref):
    agg = agg_ref[0] + agg_ref[1]
    o = (1.0 - ALPHA) * (dinv_ref[...] * agg + dinv2_ref[...] * out_ref[...])
    o = o + ALPHA * h_ref[...]
    out_next_ref[...] = o
    p_next_ref[...] = dinv_ref[...] * o


def _lsm_body(o_ref, y_ref):
    o = o_ref[...]
    m = jnp.max(o, axis=1, keepdims=True)
    y = o - m
    y_ref[...] = y - jnp.log(jnp.sum(jnp.exp(y), axis=1, keepdims=True))


def _deg_body(dst_hbm, ones_hbm, zeros_hbm, out_hbm, dst_vm, ones_vm, deg_sh):
    c = lax.axis_index("c")
    s = lax.axis_index("s")
    w = c * NS + s
    pltpu.sync_copy(zeros_hbm.at[pl.ds(s * RPT, RPT)],
                    deg_sh.at[pl.ds(s * RPT, RPT)])
    pltpu.sync_copy(ones_hbm, ones_vm)
    pltpu.sync_copy(dst_hbm.at[w], dst_vm)
    plsc.subcore_barrier()

    @pl.loop(0, CH)
    def _chunks(j):
        pltpu.sync_copy(ones_vm, deg_sh.at[dst_vm.at[j]], add=True)

    plsc.subcore_barrier()
    pltpu.sync_copy(deg_sh.at[pl.ds(s * RPT, RPT)],
                    out_hbm.at[c, pl.ds(s * RPT, RPT)])


NBUF = 8
LOOK = 4


def _spmm_body(p_hbm, src_hbm, dst_hbm, zeros_hbm, out_hbm,
               src_vm, dst_vm, bufs, agg_sh, gsems, ssems):
    c = lax.axis_index("c")
    s = lax.axis_index("s")
    w = c * NS + s
    pltpu.sync_copy(zeros_hbm.at[pl.ds(s * RPT, RPT)],
                    agg_sh.at[pl.ds(s * RPT, RPT)])
    pltpu.sync_copy(src_hbm.at[w], src_vm)
    pltpu.sync_copy(dst_hbm.at[w], dst_vm)
    plsc.subcore_barrier()

    for b in range(LOOK):
        pltpu.async_copy(p_hbm.at[src_vm.at[b]], bufs.at[b], gsems.at[b])

    @pl.loop(0, CH, step=NBUF)
    def _chunks(j):
        for b in range(NBUF):
            jj = j + b
            jl = jj + LOOK
            bl = (b + LOOK) % NBUF

            @pl.when(jl < CH)
            def _():
                @pl.when(jl >= NBUF)
                def _():
                    pltpu.make_async_copy(
                        bufs.at[bl], agg_sh.at[dst_vm.at[jl - NBUF]],
                        ssems.at[bl]).wait()
                pltpu.async_copy(p_hbm.at[src_vm.at[jl]], bufs.at[bl],
                                 gsems.at[bl])

            pltpu.make_async_copy(p_hbm.at[src_vm.at[jj]], bufs.at[b],
                                  gsems.at[b]).wait()
            pltpu.async_copy(bufs.at[b], agg_sh.at[dst_vm.at[jj]],
                             ssems.at[b], add=True)

    for b in range(NBUF):
        jj = CH - NBUF + b
        pltpu.make_async_copy(bufs.at[b], agg_sh.at[dst_vm.at[jj]],
                              ssems.at[b]).wait()

    plsc.subcore_barrier()
    pltpu.sync_copy(agg_sh.at[pl.ds(s * RPT, RPT)],
                    out_hbm.at[c, pl.ds(s * RPT, RPT)])


_mlp_call = pl.pallas_call(
    _mlp_body,
    out_shape=jax.ShapeDtypeStruct((NP, OUT_C), jnp.float32),
)

_prep_call = pl.pallas_call(
    _prep_body,
    out_shape=(
        jax.ShapeDtypeStruct((NP, 1), jnp.float32),
        jax.ShapeDtypeStruct((NP, 1), jnp.float32),
        jax.ShapeDtypeStruct((NP, OUT_C), jnp.float32),
    ),
)

_blend_call = pl.pallas_call(
    _blend_body,
    out_shape=(
        jax.ShapeDtypeStruct((NP, OUT_C), jnp.float32),
        jax.ShapeDtypeStruct((NP, OUT_C), jnp.float32),
    ),
)

_lsm_call = pl.pallas_call(
    _lsm_body,
    out_shape=jax.ShapeDtypeStruct((NP, OUT_C), jnp.float32),
)

_deg_call = pl.kernel(
    _deg_body,
    out_type=jax.ShapeDtypeStruct((NC, NP, 16), jnp.float32),
    mesh=_mesh,
    compiler_params=pltpu.CompilerParams(use_tc_tiling_on_sc=False),
    scratch_types=[
        pltpu.VMEM((CH, C), jnp.int32),
        pltpu.VMEM((C, 16), jnp.float32),
        pltpu.VMEM_SHARED((NP, 16), jnp.float32),
    ],
)

_spmm_call = pl.kernel(
    _spmm_body,
    out_type=jax.ShapeDtypeStruct((NC, NP, OUT_C), jnp.float32),
    mesh=_mesh,
    compiler_params=pltpu.CompilerParams(use_tc_tiling_on_sc=False),
    scratch_types=[
        pltpu.VMEM((CH, C), jnp.int32),
        pltpu.VMEM((CH, C), jnp.int32),
        pltpu.VMEM((NBUF, C, OUT_C), jnp.float32),
        pltpu.VMEM_SHARED((NP, OUT_C), jnp.float32),
        pltpu.SemaphoreType.DMA((NBUF,)),
        pltpu.SemaphoreType.DMA((NBUF,)),
    ],
)


def kernel(x, edge_index, W1, b1, W2, b2):
    f32 = jnp.float32
    x_pad = jnp.concatenate([x, jnp.zeros((NP - N, IN_C), f32)], axis=0)
    pad = jnp.full((E_PAD - E,), N, jnp.int32)
    srcp = jnp.concatenate([edge_index[0], pad]).reshape(W, CH, C)
    dstp = jnp.concatenate([edge_index[1], pad]).reshape(W, CH, C)

    ones16 = jnp.ones((C, 16), f32)
    zeros16 = jnp.zeros((NP, 16), f32)
    zeros64 = jnp.zeros((NP, OUT_C), f32)

    h = _mlp_call(x_pad, W1, b1.reshape(1, HID), W2, b2.reshape(1, OUT_C))
    deg16 = _deg_call(dstp, ones16, zeros16)
    dinv, dinv2, p = _prep_call(deg16, h)

    out = h
    for _ in range(K):
        agg2 = _spmm_call(p, srcp, dstp, zeros64)
        out, p = _blend_call(agg2, out, h, dinv, dinv2)

    y = _lsm_call(out)
    return y[:N]


# speedup vs baseline: 2.5521x; 2.5521x over previous
import functools

import jax
import jax.numpy as jnp
from jax import lax
from jax.experimental import pallas as pl
from jax.experimental.pallas import tpu as pltpu
from jax.experimental.pallas import tpu_sc as plsc

N = 10000
E = 320000
IN_C = 128
HID = 16
OUT_C = 64
K = 10
ALPHA = 0.1

NC = 2
NS = 16
W = NC * NS
C = 128
CH = 80
CH2 = 160
HC = OUT_C // 2
EPW = C * CH
E_PAD = W * EPW
NP = 10112
RPT = NP // NS

_mesh = plsc.VectorSubcoreMesh(
    core_axis_name="c", subcore_axis_name="s", num_cores=NC, num_subcores=NS
)


def _mlp_body(x_ref, w1_ref, b1_ref, w2_ref, b2_ref, h_ref):
    h1 = jnp.dot(x_ref[...], w1_ref[...], preferred_element_type=jnp.float32,
                 precision=lax.Precision.HIGHEST)
    h1 = jnp.maximum(h1 + b1_ref[...], 0.0)
    h2 = jnp.dot(h1, w2_ref[...], preferred_element_type=jnp.float32,
                 precision=lax.Precision.HIGHEST)
    h_ref[...] = h2 + b2_ref[...]


def _prep_body(deg16_ref, h_ref, dinv_ref, dinv2_ref, p0_ref):
    degs = deg16_ref[0] + deg16_ref[1]
    deg = jnp.sum(degs, axis=1, keepdims=True) * (1.0 / 16.0) + 1.0
    rows = lax.broadcasted_iota(jnp.int32, (NP, 1), 0)
    dinv = jnp.where(rows < N, lax.rsqrt(deg), 0.0)
    dinv_ref[...] = dinv
    dinv2_ref[...] = dinv * dinv
    p0 = dinv * h_ref[...]
    p0_ref[0] = p0[:, :HC]
    p0_ref[1] = p0[:, HC:]


def _blend_body(agg_ref, out_ref, h_ref, dinv_ref, dinv2_ref,
                out_next_ref, p_next_ref):
    agg = jnp.concatenate([agg_ref[0], agg_ref[1]], axis=1)
    o = (1.0 - ALPHA) * (dinv_ref[...] * agg + dinv2_ref[...] * out_ref[...])
    o = o + ALPHA * h_ref[...]
    out_next_ref[...] = o
    p2 = dinv_ref[...] * o
    p_next_ref[0] = p2[:, :HC]
    p_next_ref[1] = p2[:, HC:]


def _lsm_body(o_ref, y_ref):
    o = o_ref[...]
    m = jnp.max(o, axis=1, keepdims=True)
    y = o - m
    y_ref[...] = y - jnp.log(jnp.sum(jnp.exp(y), axis=1, keepdims=True))


def _deg_body(dst_hbm, ones_hbm, zeros_hbm, out_hbm, dst_vm, ones_vm, deg_sh):
    c = lax.axis_index("c")
    s = lax.axis_index("s")
    w = c * NS + s
    pltpu.sync_copy(zeros_hbm.at[pl.ds(s * RPT, RPT)],
                    deg_sh.at[pl.ds(s * RPT, RPT)])
    pltpu.sync_copy(ones_hbm, ones_vm)
    pltpu.sync_copy(dst_hbm.at[w], dst_vm)
    plsc.subcore_barrier()

    @pl.loop(0, CH)
    def _chunks(j):
        pltpu.sync_copy(ones_vm, deg_sh.at[dst_vm.at[j]], add=True)

    plsc.subcore_barrier()
    pltpu.sync_copy(deg_sh.at[pl.ds(s * RPT, RPT)],
                    out_hbm.at[c, pl.ds(s * RPT, RPT)])


NBUF = 8
LOOK = 4


def _spmm_body(p_hbm, src_hbm, dst_hbm, zeros_hbm, out_hbm,
               src_vm, dst_vm, bufs, agg_sh, p_sh, gsems, ssems):
    c = lax.axis_index("c")
    s = lax.axis_index("s")
    pltpu.sync_copy(zeros_hbm.at[pl.ds(s * RPT, RPT)],
                    agg_sh.at[pl.ds(s * RPT, RPT)])
    pltpu.sync_copy(p_hbm.at[c, pl.ds(s * RPT, RPT)],
                    p_sh.at[pl.ds(s * RPT, RPT)])
    pltpu.sync_copy(src_hbm.at[s], src_vm)
    pltpu.sync_copy(dst_hbm.at[s], dst_vm)
    plsc.subcore_barrier()

    for b in range(LOOK):
        pltpu.async_copy(p_sh.at[src_vm.at[b]], bufs.at[b], gsems.at[b])

    @pl.loop(0, CH2, step=NBUF)
    def _chunks(j):
        for b in range(NBUF):
            jj = j + b
            jl = jj + LOOK
            bl = (b + LOOK) % NBUF

            @pl.when(jl < CH2)
            def _():
                @pl.when(jl >= NBUF)
                def _():
                    pltpu.make_async_copy(
                        bufs.at[bl], agg_sh.at[dst_vm.at[jl - NBUF]],
                        ssems.at[bl]).wait()
                pltpu.async_copy(p_sh.at[src_vm.at[jl]], bufs.at[bl],
                                 gsems.at[bl])

            pltpu.make_async_copy(p_sh.at[src_vm.at[jj]], bufs.at[b],
                                  gsems.at[b]).wait()
            pltpu.async_copy(bufs.at[b], agg_sh.at[dst_vm.at[jj]],
                             ssems.at[b], add=True)

    for b in range(NBUF):
        jj = CH2 - NBUF + b
        pltpu.make_async_copy(bufs.at[b], agg_sh.at[dst_vm.at[jj]],
                              ssems.at[b]).wait()

    plsc.subcore_barrier()
    pltpu.sync_copy(agg_sh.at[pl.ds(s * RPT, RPT)],
                    out_hbm.at[c, pl.ds(s * RPT, RPT)])


_mlp_call = pl.pallas_call(
    _mlp_body,
    out_shape=jax.ShapeDtypeStruct((NP, OUT_C), jnp.float32),
)

_prep_call = pl.pallas_call(
    _prep_body,
    out_shape=(
        jax.ShapeDtypeStruct((NP, 1), jnp.float32),
        jax.ShapeDtypeStruct((NP, 1), jnp.float32),
        jax.ShapeDtypeStruct((NC, NP, HC), jnp.float32),
    ),
)

_blend_call = pl.pallas_call(
    _blend_body,
    out_shape=(
        jax.ShapeDtypeStruct((NP, OUT_C), jnp.float32),
        jax.ShapeDtypeStruct((NC, NP, HC), jnp.float32),
    ),
)

_lsm_call = pl.pallas_call(
    _lsm_body,
    out_shape=jax.ShapeDtypeStruct((NP, OUT_C), jnp.float32),
)

_deg_call = pl.kernel(
    _deg_body,
    out_type=jax.ShapeDtypeStruct((NC, NP, 16), jnp.float32),
    mesh=_mesh,
    compiler_params=pltpu.CompilerParams(use_tc_tiling_on_sc=False),
    scratch_types=[
        pltpu.VMEM((CH, C), jnp.int32),
        pltpu.VMEM((C, 16), jnp.float32),
        pltpu.VMEM_SHARED((NP, 16), jnp.float32),
    ],
)

_spmm_call = pl.kernel(
    _spmm_body,
    out_type=jax.ShapeDtypeStruct((NC, NP, HC), jnp.float32),
    mesh=_mesh,
    compiler_params=pltpu.CompilerParams(use_tc_tiling_on_sc=False),
    scratch_types=[
        pltpu.VMEM((CH2, C), jnp.int32),
        pltpu.VMEM((CH2, C), jnp.int32),
        pltpu.VMEM((NBUF, C, HC), jnp.float32),
        pltpu.VMEM_SHARED((NP, HC), jnp.float32),
        pltpu.VMEM_SHARED((NP, HC), jnp.float32),
        pltpu.SemaphoreType.DMA((NBUF,)),
        pltpu.SemaphoreType.DMA((NBUF,)),
    ],
)


def kernel(x, edge_index, W1, b1, W2, b2):
    f32 = jnp.float32
    x_pad = jnp.concatenate([x, jnp.zeros((NP - N, IN_C), f32)], axis=0)
    pad = jnp.full((E_PAD - E,), N, jnp.int32)
    srcp = jnp.concatenate([edge_index[0], pad]).reshape(NS, CH2, C)
    dstp = jnp.concatenate([edge_index[1], pad]).reshape(NS, CH2, C)
    dstd = jnp.concatenate([edge_index[1], pad]).reshape(W, CH, C)

    ones16 = jnp.ones((C, 16), f32)
    zeros16 = jnp.zeros((NP, 16), f32)
    zeros32 = jnp.zeros((NP, HC), f32)

    h = _mlp_call(x_pad, W1, b1.reshape(1, HID), W2, b2.reshape(1, OUT_C))
    deg16 = _deg_call(dstd, ones16, zeros16)
    dinv, dinv2, p = _prep_call(deg16, h)

    out = h
    for _ in range(K):
        agg2 = _spmm_call(p, srcp, dstp, zeros32)
        out, p = _blend_call(agg2, out, h, dinv, dinv2)

    y = _lsm_call(out)
    return y[:N]


# speedup vs baseline: 2.6965x; 1.0566x over previous
import jax
import jax.numpy as jnp
from jax import lax
from jax.experimental import pallas as pl
from jax.experimental.pallas import tpu as pltpu
from jax.experimental.pallas import tpu_sc as plsc

N = 10000
E = 320000
IN_C = 128
HID = 16
OUT_C = 64
K = 10
ALPHA = 0.1

NC = 2
NS = 16
C = 128
CH2 = 160
HC = OUT_C // 2
E_PAD = NS * CH2 * C
NP = 10240
RPT = NP // NS
SUB = RPT // C

NBUF = 8
LOOK = 4

_mesh = plsc.VectorSubcoreMesh(
    core_axis_name="c", subcore_axis_name="s", num_cores=NC, num_subcores=NS
)


def _mlp_body(x_ref, w1_ref, b1_ref, w2_ref, b2_ref, h2_ref):
    h1 = jnp.dot(x_ref[...], w1_ref[...], preferred_element_type=jnp.float32,
                 precision=lax.Precision.HIGHEST)
    h1 = jnp.maximum(h1 + b1_ref[...], 0.0)
    h2 = jnp.dot(h1, w2_ref[...], preferred_element_type=jnp.float32,
                 precision=lax.Precision.HIGHEST)
    h = h2 + b2_ref[...]
    h2_ref[0] = h[:, :HC].astype(jnp.bfloat16)
    h2_ref[1] = h[:, HC:].astype(jnp.bfloat16)


def _lsm_body(o_ref, y_ref):
    o = jnp.concatenate([o_ref[0].astype(jnp.float32),
                         o_ref[1].astype(jnp.float32)], axis=1)
    m = jnp.max(o, axis=1, keepdims=True)
    y = o - m
    y_ref[...] = y - jnp.log(jnp.sum(jnp.exp(y), axis=1, keepdims=True))


def _appnp_body(h2_hbm, ed_hbm, zeros_hbm,
                out_hbm, src_vm, dst_vm, bufs, out_vm, h_vm, dinvb_vm,
                obuf_vm, agg_sh, p_sh, gsems, ssems):
    c = lax.axis_index("c")
    s = lax.axis_index("s")
    row0 = s * RPT

    pltpu.sync_copy(ed_hbm.at[s], src_vm)
    for i in range(SUB):
        pltpu.sync_copy(zeros_hbm, agg_sh.at[pl.ds(row0 + i * C, C)])

    @pl.loop(0, CH2)
    def _unpack_edges(j):
        for cc in range(C // 16):
            cs = pl.ds(cc * 16, 16)
            v = src_vm[j, cs]
            dst_vm[j, cs] = lax.shift_right_logical(v, 16)
            src_vm[j, cs] = lax.bitwise_and(v, 0xFFFF)

    for i in range(SUB):
        pltpu.sync_copy(h2_hbm.at[c, pl.ds(row0 + i * C, C)], obuf_vm)

        @pl.loop(0, C)
        def _unpack_h(r):
            lr = i * C + r
            ha, hb = plsc.unpack(obuf_vm[r, pl.ds(0, 32)],
                                 format=plsc.PackFormat.INTERLEAVED)
            h_vm[lr, pl.ds(0, 16)] = ha
            h_vm[lr, pl.ds(16, 16)] = hb

    @pl.loop(0, C)
    def _ones(r):
        bufs[NBUF - 1, r, pl.ds(0, 16)] = jnp.full((16,), 1.0, jnp.float32)

    @pl.loop(0, RPT)
    def _init(r):
        for col in range(2):
            out_vm[r, pl.ds(col * 16, 16)] = h_vm[r, pl.ds(col * 16, 16)]

    plsc.subcore_barrier()

    @pl.loop(0, CH2, step=NBUF)
    def _deg_chunks(j):
        for b in range(NBUF):
            jj = j + b

            @pl.when(jj >= NBUF)
            def _():
                pltpu.make_async_copy(bufs.at[NBUF - 1],
                                      agg_sh.at[dst_vm.at[jj - NBUF]],
                                      ssems.at[b]).wait()
            pltpu.async_copy(bufs.at[NBUF - 1], agg_sh.at[dst_vm.at[jj]],
                             ssems.at[b], add=True)

    for b in range(NBUF):
        jj = CH2 - NBUF + b
        pltpu.make_async_copy(bufs.at[NBUF - 1], agg_sh.at[dst_vm.at[jj]],
                              ssems.at[b]).wait()

    plsc.subcore_barrier()

    for i in range(SUB):
        pltpu.sync_copy(agg_sh.at[pl.ds(row0 + i * C, C)], bufs.at[i])
        pltpu.sync_copy(zeros_hbm, agg_sh.at[pl.ds(row0 + i * C, C)])

        @pl.loop(0, C)
        def _rows(r):
            lr = i * C + r
            deg = bufs[i, r, pl.ds(0, 16)] + 1.0
            yh = plsc.bitcast(
                jnp.full((16,), 0x5F3759DF, jnp.int32)
                - lax.shift_right_logical(plsc.bitcast(deg, jnp.int32), 1),
                jnp.float32)
            half = 0.5 * deg
            for _ in range(3):
                yh = yh * (1.5 - half * yh * yh)
            live = jnp.where(row0 + lr < N, 1.0, 0.0)
            dv = yh * live
            dinvb_vm[lr, pl.ds(0, 16)] = dv
            bufs[i, r, pl.ds(0, 16)] = dv * h_vm[lr, pl.ds(0, 16)]

        pltpu.sync_copy(bufs.at[i], p_sh.at[pl.ds(row0 + i * C, C)])

    plsc.subcore_barrier()

    for q in range(2):
        qcs = pl.ds(q * 16, 16)
        if q == 1:
            for i in range(SUB):
                @pl.loop(0, C)
                def _p0(r):
                    lr = i * C + r
                    bufs[i, r, pl.ds(0, 16)] = (
                        dinvb_vm[lr, pl.ds(0, 16)] * h_vm[lr, qcs])

                pltpu.sync_copy(bufs.at[i], p_sh.at[pl.ds(row0 + i * C, C)])

            plsc.subcore_barrier()

        @pl.loop(0, K)
        def _round(_):
            for b in range(LOOK):
                pltpu.async_copy(p_sh.at[src_vm.at[b]], bufs.at[b],
                                 gsems.at[b])

            @pl.loop(0, CH2, step=NBUF)
            def _chunks(j):
                for b in range(NBUF):
                    jj = j + b
                    jl = jj + LOOK
                    bl = (b + LOOK) % NBUF

                    @pl.when(jl < CH2)
                    def _():
                        @pl.when(jl >= NBUF)
                        def _():
                            pltpu.make_async_copy(
                                bufs.at[bl], agg_sh.at[dst_vm.at[jl - NBUF]],
                                ssems.at[bl]).wait()
                        pltpu.async_copy(p_sh.at[src_vm.at[jl]], bufs.at[bl],
                                         gsems.at[bl])

                    pltpu.make_async_copy(p_sh.at[src_vm.at[jj]], bufs.at[b],
                                          gsems.at[b]).wait()
                    pltpu.async_copy(bufs.at[b], agg_sh.at[dst_vm.at[jj]],
                                     ssems.at[b], add=True)

            for b in range(NBUF):
                jj = CH2 - NBUF + b
                pltpu.make_async_copy(bufs.at[b], agg_sh.at[dst_vm.at[jj]],
                                      ssems.at[b]).wait()

            plsc.subcore_barrier()

            for i in range(SUB):
                pltpu.sync_copy(agg_sh.at[pl.ds(row0 + i * C, C)], bufs.at[i])
                pltpu.sync_copy(zeros_hbm, agg_sh.at[pl.ds(row0 + i * C, C)])

                @pl.loop(0, C)
                def _rows(r):
                    lr = i * C + r
                    dv = dinvb_vm[lr, pl.ds(0, 16)]
                    dv2 = dv * dv
                    a = bufs[i, r, pl.ds(0, 16)]
                    o = ((1.0 - ALPHA)
                         * (dv * a + dv2 * out_vm[lr, qcs])
                         + ALPHA * h_vm[lr, qcs])
                    out_vm[lr, qcs] = o
                    bufs[i, r, pl.ds(0, 16)] = dv * o

                pltpu.sync_copy(bufs.at[i], p_sh.at[pl.ds(row0 + i * C, C)])

            plsc.subcore_barrier()

    for i in range(SUB):
        @pl.loop(0, C)
        def _emit(r):
            lr = i * C + r
            ob = plsc.pack(out_vm[lr, pl.ds(0, 16)],
                           out_vm[lr, pl.ds(16, 16)],
                           format=plsc.PackFormat.INTERLEAVED)
            obuf_vm[r, pl.ds(0, 32)] = ob

        pltpu.sync_copy(obuf_vm, out_hbm.at[c, pl.ds(row0 + i * C, C)])


_mlp_call = pl.pallas_call(
    _mlp_body,
    out_shape=jax.ShapeDtypeStruct((NC, NP, HC), jnp.bfloat16),
)

_lsm_call = pl.pallas_call(
    _lsm_body,
    out_shape=jax.ShapeDtypeStruct((NP, OUT_C), jnp.float32),
)

_appnp_call = pl.kernel(
    _appnp_body,
    out_type=jax.ShapeDtypeStruct((NC, NP, HC), jnp.bfloat16),
    mesh=_mesh,
    compiler_params=pltpu.CompilerParams(use_tc_tiling_on_sc=False,
                                         needs_layout_passes=False),
    scratch_types=[
        pltpu.VMEM((CH2, C), jnp.int32),
        pltpu.VMEM((CH2, C), jnp.int32),
        pltpu.VMEM((NBUF, C, 16), jnp.float32),
        pltpu.VMEM((RPT, HC), jnp.float32),
        pltpu.VMEM((RPT, HC), jnp.float32),
        pltpu.VMEM((RPT, 16), jnp.float32),
        pltpu.VMEM((C, HC), jnp.bfloat16),
        pltpu.VMEM_SHARED((NP, 16), jnp.float32),
        pltpu.VMEM_SHARED((NP, 16), jnp.float32),
        pltpu.SemaphoreType.DMA((NBUF,)),
        pltpu.SemaphoreType.DMA((NBUF,)),
    ],
)


def kernel(x, edge_index, W1, b1, W2, b2):
    f32 = jnp.float32
    x_pad = jnp.concatenate([x, jnp.zeros((NP - N, IN_C), f32)], axis=0)
    pad = jnp.full((E_PAD - E,), N, jnp.int32)
    srcf = jnp.concatenate([edge_index[0], pad])
    dstf = jnp.concatenate([edge_index[1], pad])
    edp = jnp.bitwise_or(jnp.left_shift(dstf, 16), srcf).reshape(NS, CH2, C)

    zeros32 = jnp.zeros((C, 16), f32)

    h2 = _mlp_call(x_pad, W1, b1.reshape(1, HID), W2, b2.reshape(1, OUT_C))
    outh = _appnp_call(h2, edp, zeros32)
    y = _lsm_call(outh)
    return y[:N]


# speedup vs baseline: 2.7094x; 1.0048x over previous
import jax
import jax.numpy as jnp
from jax import lax
from jax.experimental import pallas as pl
from jax.experimental.pallas import tpu as pltpu
from jax.experimental.pallas import tpu_sc as plsc

N = 10000
E = 320000
IN_C = 128
HID = 16
OUT_C = 64
K = 10
ALPHA = 0.1

NC = 2
NS = 16
C = 128
CH2 = 160
HC = OUT_C // 2
E_PAD = NS * CH2 * C
NP = 10240
RPT = NP // NS
SUB = RPT // C

NBUF = 8
LOOK = 6

_mesh = plsc.VectorSubcoreMesh(
    core_axis_name="c", subcore_axis_name="s", num_cores=NC, num_subcores=NS
)


def _mlp_body(x_ref, w1_ref, b1_ref, w2_ref, b2_ref, h2_ref):
    h1 = jnp.dot(x_ref[...], w1_ref[...], preferred_element_type=jnp.float32,
                 precision=lax.Precision.HIGHEST)
    h1 = jnp.maximum(h1 + b1_ref[...], 0.0)
    h2 = jnp.dot(h1, w2_ref[...], preferred_element_type=jnp.float32,
                 precision=lax.Precision.HIGHEST)
    h = h2 + b2_ref[...]
    h2_ref[0] = h[:, :HC].astype(jnp.bfloat16)
    h2_ref[1] = h[:, HC:].astype(jnp.bfloat16)


def _lsm_body(o_ref, y_ref):
    o = jnp.concatenate([o_ref[0].astype(jnp.float32),
                         o_ref[1].astype(jnp.float32)], axis=1)
    m = jnp.max(o, axis=1, keepdims=True)
    y = o - m
    y_ref[...] = y - jnp.log(jnp.sum(jnp.exp(y), axis=1, keepdims=True))


def _appnp_body(h2_hbm, ed_hbm, zeros_hbm,
                out_hbm, src_vm, dst_vm, bufs, out_vm, h_vm, dinvb_vm,
                obuf_vm, agg_sh, p_sh, gsems, ssems):
    c = lax.axis_index("c")
    s = lax.axis_index("s")
    row0 = s * RPT

    pltpu.sync_copy(ed_hbm.at[s], src_vm)
    for i in range(SUB):
        pltpu.sync_copy(zeros_hbm, agg_sh.at[pl.ds(row0 + i * C, C)])

    @pl.loop(0, CH2)
    def _unpack_edges(j):
        for cc in range(C // 16):
            cs = pl.ds(cc * 16, 16)
            v = src_vm[j, cs]
            dst_vm[j, cs] = lax.shift_right_logical(v, 16)
            src_vm[j, cs] = lax.bitwise_and(v, 0xFFFF)

    for i in range(SUB):
        pltpu.sync_copy(h2_hbm.at[c, pl.ds(row0 + i * C, C)], obuf_vm)

        @pl.loop(0, C)
        def _unpack_h(r):
            lr = i * C + r
            ha, hb = plsc.unpack(obuf_vm[r, pl.ds(0, 32)],
                                 format=plsc.PackFormat.INTERLEAVED)
            h_vm[lr, pl.ds(0, 16)] = ha
            h_vm[lr, pl.ds(16, 16)] = hb

    @pl.loop(0, C)
    def _ones(r):
        bufs[NBUF - 1, r, pl.ds(0, 16)] = jnp.full((16,), 1.0, jnp.float32)

    @pl.loop(0, RPT)
    def _init(r):
        for col in range(2):
            out_vm[r, pl.ds(col * 16, 16)] = h_vm[r, pl.ds(col * 16, 16)]

    plsc.subcore_barrier()

    @pl.loop(0, CH2, step=NBUF)
    def _deg_chunks(j):
        for b in range(NBUF):
            jj = j + b

            @pl.when(jj >= NBUF)
            def _():
                pltpu.make_async_copy(bufs.at[NBUF - 1],
                                      agg_sh.at[dst_vm.at[jj - NBUF]],
                                      ssems.at[b]).wait()
            pltpu.async_copy(bufs.at[NBUF - 1], agg_sh.at[dst_vm.at[jj]],
                             ssems.at[b], add=True)

    for b in range(NBUF):
        jj = CH2 - NBUF + b
        pltpu.make_async_copy(bufs.at[NBUF - 1], agg_sh.at[dst_vm.at[jj]],
                              ssems.at[b]).wait()

    plsc.subcore_barrier()

    for i in range(SUB):
        pltpu.sync_copy(agg_sh.at[pl.ds(row0 + i * C, C)], bufs.at[i])
        pltpu.sync_copy(zeros_hbm, agg_sh.at[pl.ds(row0 + i * C, C)])

        @pl.loop(0, C)
        def _rows(r):
            lr = i * C + r
            deg = bufs[i, r, pl.ds(0, 16)] + 1.0
            yh = plsc.bitcast(
                jnp.full((16,), 0x5F3759DF, jnp.int32)
                - lax.shift_right_logical(plsc.bitcast(deg, jnp.int32), 1),
                jnp.float32)
            half = 0.5 * deg
            for _ in range(3):
                yh = yh * (1.5 - half * yh * yh)
            live = jnp.where(row0 + lr < N, 1.0, 0.0)
            dv = yh * live
            dinvb_vm[lr, pl.ds(0, 16)] = dv
            bufs[i, r, pl.ds(0, 16)] = dv * h_vm[lr, pl.ds(0, 16)]

        pltpu.sync_copy(bufs.at[i], p_sh.at[pl.ds(row0 + i * C, C)])

    plsc.subcore_barrier()

    for q in range(2):
        qcs = pl.ds(q * 16, 16)
        if q == 1:
            for i in range(SUB):
                @pl.loop(0, C)
                def _p0(r):
                    lr = i * C + r
                    bufs[i, r, pl.ds(0, 16)] = (
                        dinvb_vm[lr, pl.ds(0, 16)] * h_vm[lr, qcs])

                pltpu.sync_copy(bufs.at[i], p_sh.at[pl.ds(row0 + i * C, C)])

            plsc.subcore_barrier()

        @pl.loop(0, K)
        def _round(_):
            for b in range(LOOK):
                pltpu.async_copy(p_sh.at[src_vm.at[b]], bufs.at[b],
                                 gsems.at[b])

            @pl.loop(0, CH2, step=NBUF)
            def _chunks(j):
                for b in range(NBUF):
                    jj = j + b
                    jl = jj + LOOK
                    bl = (b + LOOK) % NBUF

                    @pl.when(jl < CH2)
                    def _():
                        @pl.when(jl >= NBUF)
                        def _():
                            pltpu.make_async_copy(
                                bufs.at[bl], agg_sh.at[dst_vm.at[jl - NBUF]],
                                ssems.at[bl]).wait()
                        pltpu.async_copy(p_sh.at[src_vm.at[jl]], bufs.at[bl],
                                         gsems.at[bl])

                    pltpu.make_async_copy(p_sh.at[src_vm.at[jj]], bufs.at[b],
                                          gsems.at[b]).wait()
                    pltpu.async_copy(bufs.at[b], agg_sh.at[dst_vm.at[jj]],
                                     ssems.at[b], add=True)

            for b in range(NBUF):
                jj = CH2 - NBUF + b
                pltpu.make_async_copy(bufs.at[b], agg_sh.at[dst_vm.at[jj]],
                                      ssems.at[b]).wait()

            plsc.subcore_barrier()

            for i in range(SUB):
                pltpu.sync_copy(agg_sh.at[pl.ds(row0 + i * C, C)], bufs.at[i])
                pltpu.sync_copy(zeros_hbm, agg_sh.at[pl.ds(row0 + i * C, C)])

                @pl.loop(0, C)
                def _rows(r):
                    lr = i * C + r
                    dv = dinvb_vm[lr, pl.ds(0, 16)]
                    dv2 = dv * dv
                    a = bufs[i, r, pl.ds(0, 16)]
                    o = ((1.0 - ALPHA)
                         * (dv * a + dv2 * out_vm[lr, qcs])
                         + ALPHA * h_vm[lr, qcs])
                    out_vm[lr, qcs] = o
                    bufs[i, r, pl.ds(0, 16)] = dv * o

                pltpu.sync_copy(bufs.at[i], p_sh.at[pl.ds(row0 + i * C, C)])

            plsc.subcore_barrier()

    for i in range(SUB):
        @pl.loop(0, C)
        def _emit(r):
            lr = i * C + r
            ob = plsc.pack(out_vm[lr, pl.ds(0, 16)],
                           out_vm[lr, pl.ds(16, 16)],
                           format=plsc.PackFormat.INTERLEAVED)
            obuf_vm[r, pl.ds(0, 32)] = ob

        pltpu.sync_copy(obuf_vm, out_hbm.at[c, pl.ds(row0 + i * C, C)])


_mlp_call = pl.pallas_call(
    _mlp_body,
    out_shape=jax.ShapeDtypeStruct((NC, NP, HC), jnp.bfloat16),
)

_lsm_call = pl.pallas_call(
    _lsm_body,
    out_shape=jax.ShapeDtypeStruct((NP, OUT_C), jnp.float32),
)

_appnp_call = pl.kernel(
    _appnp_body,
    out_type=jax.ShapeDtypeStruct((NC, NP, HC), jnp.bfloat16),
    mesh=_mesh,
    compiler_params=pltpu.CompilerParams(use_tc_tiling_on_sc=False,
                                         needs_layout_passes=False),
    scratch_types=[
        pltpu.VMEM((CH2, C), jnp.int32),
        pltpu.VMEM((CH2, C), jnp.int32),
        pltpu.VMEM((NBUF, C, 16), jnp.float32),
        pltpu.VMEM((RPT, HC), jnp.float32),
        pltpu.VMEM((RPT, HC), jnp.float32),
        pltpu.VMEM((RPT, 16), jnp.float32),
        pltpu.VMEM((C, HC), jnp.bfloat16),
        pltpu.VMEM_SHARED((NP, 16), jnp.float32),
        pltpu.VMEM_SHARED((NP, 16), jnp.float32),
        pltpu.SemaphoreType.DMA((NBUF,)),
        pltpu.SemaphoreType.DMA((NBUF,)),
    ],
)


def kernel(x, edge_index, W1, b1, W2, b2):
    f32 = jnp.float32
    x_pad = jnp.concatenate([x, jnp.zeros((NP - N, IN_C), f32)], axis=0)
    pad = jnp.full((E_PAD - E,), N, jnp.int32)
    srcf = jnp.concatenate([edge_index[0], pad])
    dstf = jnp.concatenate([edge_index[1], pad])
    edp = jnp.bitwise_or(jnp.left_shift(dstf, 16), srcf).reshape(NS, CH2, C)

    zeros32 = jnp.zeros((C, 16), f32)

    h2 = _mlp_call(x_pad, W1, b1.reshape(1, HID), W2, b2.reshape(1, OUT_C))
    outh = _appnp_call(h2, edp, zeros32)
    y = _lsm_call(outh)
    return y[:N]


# speedup vs baseline: 2.7504x; 1.0151x over previous
import jax
import jax.numpy as jnp
from jax import lax
from jax.experimental import pallas as pl
from jax.experimental.pallas import tpu as pltpu
from jax.experimental.pallas import tpu_sc as plsc

N = 10000
E = 320000
IN_C = 128
HID = 16
OUT_C = 64
K = 10
ALPHA = 0.1

NC = 2
NS = 16
C = 128
CH2 = 160
HC = OUT_C // 2
E_PAD = NS * CH2 * C
NP = 10240
RPT = NP // NS
SUB = RPT // C

W = NC * NS
CHD = E_PAD // (W * C)

NBUF = 8
LOOK = 6

_mesh = plsc.VectorSubcoreMesh(
    core_axis_name="c", subcore_axis_name="s", num_cores=NC, num_subcores=NS
)


def _mlp_body(x_ref, w1_ref, b1_ref, w2_ref, b2_ref, h2_ref):
    h1 = jnp.dot(x_ref[...], w1_ref[...], preferred_element_type=jnp.float32,
                 precision=lax.Precision.HIGHEST)
    h1 = jnp.maximum(h1 + b1_ref[...], 0.0)
    h2 = jnp.dot(h1, w2_ref[...], preferred_element_type=jnp.float32,
                 precision=lax.Precision.HIGHEST)
    h = h2 + b2_ref[...]
    h2_ref[0] = h[:, :HC].astype(jnp.bfloat16)
    h2_ref[1] = h[:, HC:].astype(jnp.bfloat16)


def _lsm_body(o_ref, y_ref):
    o = jnp.concatenate([o_ref[0].astype(jnp.float32),
                         o_ref[1].astype(jnp.float32)], axis=1)
    m = jnp.max(o, axis=1, keepdims=True)
    y = o - m
    y_ref[...] = y - jnp.log(jnp.sum(jnp.exp(y), axis=1, keepdims=True))


def _deg_body(dst_hbm, ones_hbm, zeros_hbm, out_hbm, dst_vm, ones_vm, deg_sh):
    c = lax.axis_index("c")
    s = lax.axis_index("s")
    w = c * NS + s
    pltpu.sync_copy(zeros_hbm.at[pl.ds(s * RPT, RPT)],
                    deg_sh.at[pl.ds(s * RPT, RPT)])
    pltpu.sync_copy(ones_hbm, ones_vm)
    pltpu.sync_copy(dst_hbm.at[w], dst_vm)
    plsc.subcore_barrier()

    @pl.loop(0, CHD)
    def _chunks(j):
        pltpu.sync_copy(ones_vm, deg_sh.at[dst_vm.at[j]], add=True)

    plsc.subcore_barrier()
    pltpu.sync_copy(deg_sh.at[pl.ds(s * RPT, RPT)],
                    out_hbm.at[c, pl.ds(s * RPT, RPT)])


def _appnp_body(deg16_hbm, h2_hbm, ed_hbm, zeros_hbm,
                out_hbm, src_vm, dst_vm, bufs, out_vm, h_vm, dinvb_vm,
                obuf_vm, agg_sh, p_sh, gsems, ssems):
    c = lax.axis_index("c")
    s = lax.axis_index("s")
    row0 = s * RPT

    pltpu.sync_copy(ed_hbm.at[s], src_vm)
    for i in range(SUB):
        pltpu.sync_copy(zeros_hbm, agg_sh.at[pl.ds(row0 + i * C, C)])

    @pl.loop(0, CH2)
    def _unpack_edges(j):
        for cc in range(C // 16):
            cs = pl.ds(cc * 16, 16)
            v = src_vm[j, cs]
            dst_vm[j, cs] = lax.shift_right_logical(v, 16)
            src_vm[j, cs] = lax.bitwise_and(v, 0xFFFF)

    for i in range(SUB):
        pltpu.sync_copy(h2_hbm.at[c, pl.ds(row0 + i * C, C)], obuf_vm)

        @pl.loop(0, C)
        def _unpack_h(r):
            lr = i * C + r
            ha, hb = plsc.unpack(obuf_vm[r, pl.ds(0, 32)],
                                 format=plsc.PackFormat.INTERLEAVED)
            h_vm[lr, pl.ds(0, 16)] = ha
            h_vm[lr, pl.ds(16, 16)] = hb

    @pl.loop(0, RPT)
    def _init(r):
        for col in range(2):
            out_vm[r, pl.ds(col * 16, 16)] = h_vm[r, pl.ds(col * 16, 16)]

    for i in range(SUB):
        pltpu.sync_copy(deg16_hbm.at[0, pl.ds(row0 + i * C, C)],
                        bufs.at[2 * (i % 4)])
        pltpu.sync_copy(deg16_hbm.at[1, pl.ds(row0 + i * C, C)],
                        bufs.at[2 * (i % 4) + 1])

        @pl.loop(0, C)
        def _rows(r):
            lr = i * C + r
            deg = (bufs[2 * (i % 4), r, pl.ds(0, 16)]
                   + bufs[2 * (i % 4) + 1, r, pl.ds(0, 16)] + 1.0)
            yh = plsc.bitcast(
                jnp.full((16,), 0x5F3759DF, jnp.int32)
                - lax.shift_right_logical(plsc.bitcast(deg, jnp.int32), 1),
                jnp.float32)
            half = 0.5 * deg
            for _ in range(3):
                yh = yh * (1.5 - half * yh * yh)
            live = jnp.where(row0 + lr < N, 1.0, 0.0)
            dv = yh * live
            dinvb_vm[lr, pl.ds(0, 16)] = dv
            bufs[2 * (i % 4), r, pl.ds(0, 16)] = dv * h_vm[lr, pl.ds(0, 16)]

        pltpu.sync_copy(bufs.at[2 * (i % 4)],
                        p_sh.at[pl.ds(row0 + i * C, C)])

    plsc.subcore_barrier()

    for q in range(2):
        qcs = pl.ds(q * 16, 16)
        if q == 1:
            for i in range(SUB):
                @pl.loop(0, C)
                def _p0(r):
                    lr = i * C + r
                    bufs[i, r, pl.ds(0, 16)] = (
                        dinvb_vm[lr, pl.ds(0, 16)] * h_vm[lr, qcs])

                pltpu.sync_copy(bufs.at[i], p_sh.at[pl.ds(row0 + i * C, C)])

            plsc.subcore_barrier()

        @pl.loop(0, K)
        def _round(_):
            for b in range(LOOK):
                pltpu.async_copy(p_sh.at[src_vm.at[b]], bufs.at[b],
                                 gsems.at[b])

            @pl.loop(0, CH2, step=NBUF)
            def _chunks(j):
                for b in range(NBUF):
                    jj = j + b
                    jl = jj + LOOK
                    bl = (b + LOOK) % NBUF

                    @pl.when(jl < CH2)
                    def _():
                        @pl.when(jl >= NBUF)
                        def _():
                            pltpu.make_async_copy(
                                bufs.at[bl], agg_sh.at[dst_vm.at[jl - NBUF]],
                                ssems.at[bl]).wait()
                        pltpu.async_copy(p_sh.at[src_vm.at[jl]], bufs.at[bl],
                                         gsems.at[bl])

                    pltpu.make_async_copy(p_sh.at[src_vm.at[jj]], bufs.at[b],
                                          gsems.at[b]).wait()
                    pltpu.async_copy(bufs.at[b], agg_sh.at[dst_vm.at[jj]],
                                     ssems.at[b], add=True)

            for b in range(NBUF):
                jj = CH2 - NBUF + b
                pltpu.make_async_copy(bufs.at[b], agg_sh.at[dst_vm.at[jj]],
                                      ssems.at[b]).wait()

            plsc.subcore_barrier()

            for i in range(SUB):
                pltpu.sync_copy(agg_sh.at[pl.ds(row0 + i * C, C)], bufs.at[i])
                pltpu.sync_copy(zeros_hbm, agg_sh.at[pl.ds(row0 + i * C, C)])

                @pl.loop(0, C)
                def _rows(r):
                    lr = i * C + r
                    dv = dinvb_vm[lr, pl.ds(0, 16)]
                    dv2 = dv * dv
                    a = bufs[i, r, pl.ds(0, 16)]
                    o = ((1.0 - ALPHA)
                         * (dv * a + dv2 * out_vm[lr, qcs])
                         + ALPHA * h_vm[lr, qcs])
                    out_vm[lr, qcs] = o
                    bufs[i, r, pl.ds(0, 16)] = dv * o

                pltpu.sync_copy(bufs.at[i], p_sh.at[pl.ds(row0 + i * C, C)])

            plsc.subcore_barrier()

    for i in range(SUB):
        @pl.loop(0, C)
        def _emit(r):
            lr = i * C + r
            ob = plsc.pack(out_vm[lr, pl.ds(0, 16)],
                           out_vm[lr, pl.ds(16, 16)],
                           format=plsc.PackFormat.INTERLEAVED)
            obuf_vm[r, pl.ds(0, 32)] = ob

        pltpu.sync_copy(obuf_vm, out_hbm.at[c, pl.ds(row0 + i * C, C)])


_mlp_call = pl.pallas_call(
    _mlp_body,
    out_shape=jax.ShapeDtypeStruct((NC, NP, HC), jnp.bfloat16),
)

_lsm_call = pl.pallas_call(
    _lsm_body,
    out_shape=jax.ShapeDtypeStruct((NP, OUT_C), jnp.float32),
)

_deg_call = pl.kernel(
    _deg_body,
    out_type=jax.ShapeDtypeStruct((NC, NP, 16), jnp.float32),
    mesh=_mesh,
    compiler_params=pltpu.CompilerParams(use_tc_tiling_on_sc=False),
    scratch_types=[
        pltpu.VMEM((CHD, C), jnp.int32),
        pltpu.VMEM((C, 16), jnp.float32),
        pltpu.VMEM_SHARED((NP, 16), jnp.float32),
    ],
)

_appnp_call = pl.kernel(
    _appnp_body,
    out_type=jax.ShapeDtypeStruct((NC, NP, HC), jnp.bfloat16),
    mesh=_mesh,
    compiler_params=pltpu.CompilerParams(use_tc_tiling_on_sc=False,
                                         needs_layout_passes=False),
    scratch_types=[
        pltpu.VMEM((CH2, C), jnp.int32),
        pltpu.VMEM((CH2, C), jnp.int32),
        pltpu.VMEM((NBUF, C, 16), jnp.float32),
        pltpu.VMEM((RPT, HC), jnp.float32),
        pltpu.VMEM((RPT, HC), jnp.float32),
        pltpu.VMEM((RPT, 16), jnp.float32),
        pltpu.VMEM((C, HC), jnp.bfloat16),
        pltpu.VMEM_SHARED((NP, 16), jnp.float32),
        pltpu.VMEM_SHARED((NP, 16), jnp.float32),
        pltpu.SemaphoreType.DMA((NBUF,)),
        pltpu.SemaphoreType.DMA((NBUF,)),
    ],
)


def kernel(x, edge_index, W1, b1, W2, b2):
    f32 = jnp.float32
    x_pad = jnp.concatenate([x, jnp.zeros((NP - N, IN_C), f32)], axis=0)
    pad = jnp.full((E_PAD - E,), N, jnp.int32)
    srcf = jnp.concatenate([edge_index[0], pad])
    dstf = jnp.concatenate([edge_index[1], pad])
    edp = jnp.bitwise_or(jnp.left_shift(dstf, 16), srcf).reshape(NS, CH2, C)

    zeros32 = jnp.zeros((C, 16), f32)
    ones16 = jnp.ones((C, 16), f32)
    zeros_np16 = jnp.zeros((NP, 16), f32)
    dstd = dstf.reshape(W, CHD, C)

    deg16 = _deg_call(dstd, ones16, zeros_np16)
    h2 = _mlp_call(x_pad, W1, b1.reshape(1, HID), W2, b2.reshape(1, OUT_C))
    outh = _appnp_call(deg16, h2, edp, zeros32)
    y = _lsm_call(outh)
    return y[:N]
